# Initial kernel scaffold; baseline (speedup 1.0000x reference)
import functools

import jax
import jax.numpy as jnp
from jax import lax
from jax.experimental import pallas as pl
from jax.experimental.pallas import tpu as pltpu
from jax.experimental.pallas import tpu_sc as plsc

N = 10000
E = 320000
D = 128

NC = 2
NS = 16

EPT = E // NS
CH = 80
NCHUNK = EPT // CH
ROWS_PT = N // NS
RB = 125
NRB = ROWS_PT // RB

_mesh = plsc.VectorSubcoreMesh(
    core_axis_name="c", subcore_axis_name="s", num_cores=NC, num_subcores=NS)


@functools.partial(
    pl.kernel,
    out_type=(
        jax.ShapeDtypeStruct((N, D), jnp.float32),
        jax.ShapeDtypeStruct((N, D), jnp.float32),
    ),
    mesh=_mesh,
    scratch_types=[
        pltpu.VMEM_SHARED((N, D), jnp.float32),
        pltpu.VMEM((CH,), jnp.int32),
        pltpu.VMEM((CH,), jnp.int32),
        pltpu.VMEM((CH, D), jnp.float32),
        pltpu.VMEM((RB, D), jnp.float32),
        pltpu.SemaphoreType.DMA,
    ],
)
def _h2gcn_sc(x_hbm, edges_hbm, zeros_hbm, out1_hbm, out2_hbm,
              acc, src_idx, dst_idx, rows, obuf, sem):
    c = lax.axis_index("c")
    s = lax.axis_index("s")

    def zbody(i, _):
        base = s * ROWS_PT + i * RB
        pltpu.sync_copy(zeros_hbm.at[pl.ds(base, RB)], obuf)
        pltpu.sync_copy(obuf, acc.at[pl.ds(base, RB)])
        return 0
    lax.fori_loop(0, NRB, zbody, 0)
    plsc.subcore_barrier()

    src_off = (2 * c) * E
    dst_off = (2 * c + 1) * E

    def ebody(j, _):
        base = s * EPT + j * CH
        pltpu.sync_copy(edges_hbm.at[pl.ds(src_off + base, CH)], src_idx)
        pltpu.sync_copy(edges_hbm.at[pl.ds(dst_off + base, CH)], dst_idx)
        pltpu.async_copy(x_hbm.at[src_idx], rows, sem).wait()
        pltpu.sync_copy(rows, acc.at[dst_idx], add=True)
        return 0
    lax.fori_loop(0, NCHUNK, ebody, 0)
    plsc.subcore_barrier()

    def obody(i, _):
        base = s * ROWS_PT + i * RB
        pltpu.sync_copy(acc.at[pl.ds(base, RB)], obuf)

        @pl.when(c == 0)
        def _():
            pltpu.sync_copy(obuf, out1_hbm.at[pl.ds(base, RB)])

        @pl.when(c == 1)
        def _():
            pltpu.sync_copy(obuf, out2_hbm.at[pl.ds(base, RB)])
        return 0
    lax.fori_loop(0, NRB, obody, 0)


def kernel(x, edge_index, edge_index2):
    edges = jnp.concatenate(
        [edge_index.astype(jnp.int32).reshape(-1),
         edge_index2.astype(jnp.int32).reshape(-1)])
    zeros = jnp.zeros((N, D), jnp.float32)
    x1, x2 = _h2gcn_sc(x, edges, zeros)
    return jnp.concatenate([x1, x2], axis=1)


# speedup vs baseline: 4.3798x; 4.3798x over previous
import functools

import jax
import jax.numpy as jnp
from jax import lax
from jax.experimental import pallas as pl
from jax.experimental.pallas import tpu as pltpu
from jax.experimental.pallas import tpu_sc as plsc

N = 10000
E = 320000
D = 128

NC = 2
NS = 16

EPT = E // NS
CH = 80
NCHUNK = EPT // CH
RB = 80
NRC = N // RB
RC_PER_TILE = -(-NRC // NS)

_mesh = plsc.VectorSubcoreMesh(
    core_axis_name="c", subcore_axis_name="s", num_cores=NC, num_subcores=NS)


@functools.partial(
    pl.kernel,
    out_type=(
        jax.ShapeDtypeStruct((N, D), jnp.float32),
        jax.ShapeDtypeStruct((N, D), jnp.float32),
    ),
    mesh=_mesh,
    scratch_types=[
        pltpu.VMEM_SHARED((N, D), jnp.float32),
        pltpu.VMEM((CH,), jnp.int32),
        pltpu.VMEM((CH,), jnp.int32),
        pltpu.VMEM((CH, D), jnp.float32),
        pltpu.SemaphoreType.DMA,
    ],
)
def _h2gcn_sc(x_hbm, edges_hbm, zeros_hbm, out1_hbm, out2_hbm,
              acc, src_idx, dst_idx, rows, sem):
    c = lax.axis_index("c")
    s = lax.axis_index("s")

    def zbody(i, _):
        k = i * NS + s

        @pl.when(k < NRC)
        def _():
            base = k * RB
            pltpu.sync_copy(zeros_hbm.at[pl.ds(base, RB)], rows)
            pltpu.sync_copy(rows, acc.at[pl.ds(base, RB)])
        return 0
    lax.fori_loop(0, RC_PER_TILE, zbody, 0)
    plsc.subcore_barrier()

    src_off = (2 * c) * E
    dst_off = (2 * c + 1) * E

    def ebody(j, _):
        base = s * EPT + j * CH
        pltpu.sync_copy(edges_hbm.at[pl.ds(src_off + base, CH)], src_idx)
        pltpu.sync_copy(edges_hbm.at[pl.ds(dst_off + base, CH)], dst_idx)
        pltpu.async_copy(x_hbm.at[src_idx], rows, sem).wait()
        pltpu.sync_copy(rows, acc.at[dst_idx], add=True)
        return 0
    lax.fori_loop(0, NCHUNK, ebody, 0)
    plsc.subcore_barrier()

    def obody(i, _):
        k = i * NS + s

        @pl.when(k < NRC)
        def _():
            base = k * RB
            pltpu.sync_copy(acc.at[pl.ds(base, RB)], rows)

            @pl.when(c == 0)
            def _():
                pltpu.sync_copy(rows, out1_hbm.at[pl.ds(base, RB)])

            @pl.when(c == 1)
            def _():
                pltpu.sync_copy(rows, out2_hbm.at[pl.ds(base, RB)])
        return 0
    lax.fori_loop(0, RC_PER_TILE, obody, 0)


def kernel(x, edge_index, edge_index2):
    edges = jnp.concatenate(
        [edge_index.astype(jnp.int32).reshape(-1),
         edge_index2.astype(jnp.int32).reshape(-1)])
    zeros = jnp.zeros((N, D), jnp.float32)
    x1, x2 = _h2gcn_sc(x, edges, zeros)
    return jnp.concatenate([x1, x2], axis=1)


# speedup vs baseline: 10.2639x; 2.3435x over previous
import functools

import jax
import jax.numpy as jnp
from jax import lax
from jax.experimental import pallas as pl
from jax.experimental.pallas import tpu as pltpu
from jax.experimental.pallas import tpu_sc as plsc

N = 10000
E = 320000
D = 128

NC = 2
NS = 16

EPT = E // NS
CH = 80
NCHUNK = EPT // CH
NBUF = 4
NGRP = -(-NCHUNK // NBUF)

RB = 80
NRC = N // RB
RC_PER_TILE = -(-NRC // NS)

_mesh = plsc.VectorSubcoreMesh(
    core_axis_name="c", subcore_axis_name="s", num_cores=NC, num_subcores=NS)


@functools.partial(
    pl.kernel,
    out_type=(
        jax.ShapeDtypeStruct((N, D), jnp.float32),
        jax.ShapeDtypeStruct((N, D), jnp.float32),
    ),
    mesh=_mesh,
    scratch_types=[
        pltpu.VMEM_SHARED((N, D), jnp.float32),
        [pltpu.VMEM((CH,), jnp.int32)] * NBUF,
        [pltpu.VMEM((CH,), jnp.int32)] * NBUF,
        [pltpu.VMEM((CH, D), jnp.float32)] * NBUF,
        [pltpu.SemaphoreType.DMA] * NBUF,
        [pltpu.SemaphoreType.DMA] * NBUF,
        [pltpu.SemaphoreType.DMA] * NBUF,
    ],
)
def _h2gcn_sc(x_hbm, edges_hbm, zeros_hbm, out1_hbm, out2_hbm,
              acc, srcb, dstb, rows, isem, gsem, ssem):
    c = lax.axis_index("c")
    s = lax.axis_index("s")

    def zbody(i, _):
        k = i * NS + s

        @pl.when(k < NRC)
        def _():
            base = k * RB
            pltpu.sync_copy(zeros_hbm.at[pl.ds(base, RB)], rows[0])
            pltpu.sync_copy(rows[0], acc.at[pl.ds(base, RB)])
        return 0
    lax.fori_loop(0, RC_PER_TILE, zbody, 0)
    plsc.subcore_barrier()

    tb = s * EPT
    src_off = (2 * c) * E + tb
    dst_off = (2 * c + 1) * E + tb

    def fetch(j, b):
        off = j * CH
        pltpu.async_copy(edges_hbm.at[pl.ds(src_off + off, CH)],
                         srcb[b], isem[b])
        pltpu.async_copy(edges_hbm.at[pl.ds(dst_off + off, CH)],
                         dstb[b], isem[b])

    for b in range(NBUF):
        fetch(b, b)
        pltpu.make_async_copy(edges_hbm.at[pl.ds(src_off, CH)],
                              srcb[b], isem[b]).wait()
        pltpu.make_async_copy(edges_hbm.at[pl.ds(dst_off, CH)],
                              dstb[b], isem[b]).wait()
        pltpu.async_copy(x_hbm.at[srcb[b]], rows[b], gsem[b])

    @pl.loop(0, NGRP)
    def _(g):
        base = g * NBUF
        for b in range(NBUF):
            j = base + b

            @pl.when(j < NCHUNK)
            def _():
                pltpu.make_async_copy(x_hbm.at[srcb[b]], rows[b],
                                      gsem[b]).wait()
                pltpu.async_copy(rows[b], acc.at[dstb[b]], ssem[b], add=True)
        for b in range(NBUF):
            j = base + b
            j2 = j + NBUF

            @pl.when(j < NCHUNK)
            def _():
                pltpu.make_async_copy(rows[b], acc.at[dstb[b]],
                                      ssem[b]).wait()

            @pl.when(j2 < NCHUNK)
            def _():
                fetch(j2, b)
                pltpu.make_async_copy(
                    edges_hbm.at[pl.ds(src_off, CH)], srcb[b], isem[b]).wait()
                pltpu.make_async_copy(
                    edges_hbm.at[pl.ds(dst_off, CH)], dstb[b], isem[b]).wait()
                pltpu.async_copy(x_hbm.at[srcb[b]], rows[b], gsem[b])

    plsc.subcore_barrier()

    def obody(i, _):
        k = i * NS + s

        @pl.when(k < NRC)
        def _():
            base = k * RB
            pltpu.sync_copy(acc.at[pl.ds(base, RB)], rows[0])

            @pl.when(c == 0)
            def _():
                pltpu.sync_copy(rows[0], out1_hbm.at[pl.ds(base, RB)])

            @pl.when(c == 1)
            def _():
                pltpu.sync_copy(rows[0], out2_hbm.at[pl.ds(base, RB)])
        return 0
    lax.fori_loop(0, RC_PER_TILE, obody, 0)


def kernel(x, edge_index, edge_index2):
    ei1 = edge_index.astype(jnp.int32)
    ei2 = edge_index2.astype(jnp.int32)
    edges = jnp.concatenate([ei1[0], ei1[1], ei2[0], ei2[1]])
    zeros = jnp.zeros((N, D), jnp.float32)
    x1, x2 = _h2gcn_sc(x, edges, zeros)
    return jnp.concatenate([x1, x2], axis=1)
